# Initial kernel scaffold; baseline (speedup 1.0000x reference)
import jax
import jax.numpy as jnp
from jax.experimental import pallas as pl


def kernel(x, edge_index, W1, a_src1, a_dst1, b1, W2, a_src2, a_dst2, b2, bn_c1_g, bn_c1_b, bn_c2_g, bn_c2_b, Wl1, bl1, bn1_g, bn1_b, Wl2, bl2, bn2_g, bn2_b, Wf, bf):
    raise NotImplementedError("write your pallas kernel here")


# speedup vs baseline: 31.5477x; 31.5477x over previous
import functools

import jax
import jax.numpy as jnp
import numpy as np
from jax import lax
from jax.experimental import pallas as pl
from jax.experimental.pallas import tpu as pltpu
from jax.experimental.pallas import tpu_sc as plsc

N = 10000
NP = 10240
E = 320000
F_IN = 128
NH = 64
LW = 16
NC = 2
NS = 16
EPC = E // NC
EPT = EPC // NS
CH = 80
NCHUNK = EPT // CH
RPT = NP // NS
ZROWS = 128

_f32 = jnp.float32


def _tc_pre1(x, w1p, asrc_p, adst_p):
    blk = 1000

    def body(x_ref, w_ref, as_ref, ad_ref, h_ref, s_ref, d_ref, c_ref, mx_ref):
        i = pl.program_id(0)
        h = jnp.dot(x_ref[...], w_ref[...], preferred_element_type=_f32)
        h_ref[...] = h
        s = jnp.dot(h, as_ref[...], preferred_element_type=_f32)
        d = jnp.dot(h, ad_ref[...], preferred_element_type=_f32)
        s_ref[...] = s
        d_ref[...] = d
        m = jnp.concatenate([jnp.max(s, axis=0, keepdims=True),
                             jnp.max(d, axis=0, keepdims=True)], axis=0)

        @pl.when(i == 0)
        def _():
            mx_ref[...] = m

        @pl.when(i > 0)
        def _():
            mx_ref[...] = jnp.maximum(mx_ref[...], m)

        c_ref[...] = mx_ref[0:1] + mx_ref[1:2]

    return pl.pallas_call(
        body,
        grid=(N // blk,),
        in_specs=[
            pl.BlockSpec((blk, F_IN), lambda i: (i, 0)),
            pl.BlockSpec((F_IN, NH), lambda i: (0, 0)),
            pl.BlockSpec((NH, LW), lambda i: (0, 0)),
            pl.BlockSpec((NH, LW), lambda i: (0, 0)),
        ],
        out_specs=[
            pl.BlockSpec((blk, NH), lambda i: (i, 0)),
            pl.BlockSpec((blk, LW), lambda i: (i, 0)),
            pl.BlockSpec((blk, LW), lambda i: (i, 0)),
            pl.BlockSpec((1, LW), lambda i: (0, 0)),
        ],
        out_shape=[
            jax.ShapeDtypeStruct((N, NH), _f32),
            jax.ShapeDtypeStruct((N, LW), _f32),
            jax.ShapeDtypeStruct((N, LW), _f32),
            jax.ShapeDtypeStruct((1, LW), _f32),
        ],
        scratch_shapes=[pltpu.VMEM((2, LW), _f32)],
    )(x, w1p, asrc_p, adst_p)


def _tc_rec(denp):
    blk = 1024

    def body(d_ref, r_ref):
        r_ref[...] = 1.0 / (d_ref[0] + d_ref[1] + 1e-16)

    return pl.pallas_call(
        body,
        grid=(NP // blk,),
        in_specs=[pl.BlockSpec((NC, blk, LW), lambda i: (0, i, 0))],
        out_specs=pl.BlockSpec((blk, LW), lambda i: (i, 0)),
        out_shape=jax.ShapeDtypeStruct((NP, LW), _f32),
    )(denp)


def _tc_mid(out1p, b1p, g1p, be1p, w2p, as2p, ad2p):
    blk = 1024
    ibn = 1.0 / np.sqrt(1.0 + 1e-5)

    def body(o_ref, b_ref, g_ref, be_ref, w_ref, as_ref, ad_ref,
             h_ref, s_ref, d_ref, c_ref, mx_ref):
        i = pl.program_id(0)
        z = o_ref[0] + o_ref[1] + b_ref[...]
        z = jax.nn.relu(z * ibn * g_ref[...] + be_ref[...])
        h = jnp.dot(z, w_ref[...], preferred_element_type=_f32)
        h_ref[...] = h
        s = jnp.dot(h, as_ref[...], preferred_element_type=_f32)
        d = jnp.dot(h, ad_ref[...], preferred_element_type=_f32)
        s_ref[...] = s
        d_ref[...] = d
        m = jnp.concatenate([jnp.max(s, axis=0, keepdims=True),
                             jnp.max(d, axis=0, keepdims=True)], axis=0)

        @pl.when(i == 0)
        def _():
            mx_ref[...] = m

        @pl.when(i > 0)
        def _():
            mx_ref[...] = jnp.maximum(mx_ref[...], m)

        c_ref[...] = mx_ref[0:1] + mx_ref[1:2]

    return pl.pallas_call(
        body,
        grid=(NP // blk,),
        in_specs=[
            pl.BlockSpec((NC, blk, NH), lambda i: (0, i, 0)),
            pl.BlockSpec((1, NH), lambda i: (0, 0)),
            pl.BlockSpec((1, NH), lambda i: (0, 0)),
            pl.BlockSpec((1, NH), lambda i: (0, 0)),
            pl.BlockSpec((NH, NH), lambda i: (0, 0)),
            pl.BlockSpec((NH, LW), lambda i: (0, 0)),
            pl.BlockSpec((NH, LW), lambda i: (0, 0)),
        ],
        out_specs=[
            pl.BlockSpec((blk, NH), lambda i: (i, 0)),
            pl.BlockSpec((blk, LW), lambda i: (i, 0)),
            pl.BlockSpec((blk, LW), lambda i: (i, 0)),
            pl.BlockSpec((1, LW), lambda i: (0, 0)),
        ],
        out_shape=[
            jax.ShapeDtypeStruct((NP, NH), _f32),
            jax.ShapeDtypeStruct((NP, LW), _f32),
            jax.ShapeDtypeStruct((NP, LW), _f32),
            jax.ShapeDtypeStruct((1, LW), _f32),
        ],
        scratch_shapes=[pltpu.VMEM((2, LW), _f32)],
    )(out1p, b1p, g1p, be1p, w2p, as2p, ad2p)


def _tc_head(out2p, b2, cg, cb, wl1, bl1, g1, be1, wl2, bl2, g2, be2, wf, bf):
    blk = 1024
    ibn = 1.0 / np.sqrt(1.0 + 1e-5)

    def body(o_ref, b2_ref, cg_ref, cb_ref, w1_ref, b1_ref, g1_ref, be1_ref,
             w2_ref, b2b_ref, g2_ref, be2_ref, wf_ref, bf_ref, y_ref):
        g = o_ref[0] + o_ref[1] + b2_ref[...]
        g = g * ibn * cg_ref[...] + cb_ref[...]
        t = jnp.dot(g, w1_ref[...], preferred_element_type=_f32) + b1_ref[...]
        t = jax.nn.relu(t * ibn * g1_ref[...] + be1_ref[...])
        t = jnp.dot(t, w2_ref[...], preferred_element_type=_f32) + b2b_ref[...]
        t = jax.nn.relu(t * ibn * g2_ref[...] + be2_ref[...])
        y = jnp.dot(t, wf_ref[...], preferred_element_type=_f32) + bf_ref[...]
        y_ref[...] = jax.nn.sigmoid(y)

    vec = lambda: pl.BlockSpec((1, NH), lambda i: (0, 0))
    return pl.pallas_call(
        body,
        grid=(NP // blk,),
        in_specs=[
            pl.BlockSpec((NC, blk, NH), lambda i: (0, i, 0)),
            vec(), vec(), vec(),
            pl.BlockSpec((NH, NH), lambda i: (0, 0)),
            vec(), vec(), vec(),
            pl.BlockSpec((NH, NH), lambda i: (0, 0)),
            vec(), vec(), vec(),
            pl.BlockSpec((NH, 1), lambda i: (0, 0)),
            pl.BlockSpec((1, 1), lambda i: (0, 0)),
        ],
        out_specs=pl.BlockSpec((blk, 1), lambda i: (i, 0)),
        out_shape=jax.ShapeDtypeStruct((NP, 1), _f32),
    )(out2p, b2, cg, cb, wl1, bl1, g1, be1, wl2, bl2, g2, be2, wf, bf)


def _sc_mesh():
    return plsc.VectorSubcoreMesh(core_axis_name="c", subcore_axis_name="s")


def _sc_pass1(srcv, dstv, s_tab, d_tab, cvec_hbm):
    @functools.partial(
        pl.kernel,
        mesh=_sc_mesh(),
        compiler_params=pltpu.CompilerParams(use_tc_tiling_on_sc=False),
        out_type=[
            jax.ShapeDtypeStruct((E, LW), _f32),
            jax.ShapeDtypeStruct((NC, NP, LW), _f32),
        ],
        scratch_types=[
            pltpu.VMEM((CH,), jnp.int32),
            pltpu.VMEM((CH,), jnp.int32),
            pltpu.VMEM((CH, LW), _f32),
            pltpu.VMEM((CH, LW), _f32),
            pltpu.VMEM((CH, LW), _f32),
            pltpu.VMEM((LW,), _f32),
            pltpu.VMEM((ZROWS, LW), _f32),
            pltpu.VMEM_SHARED((NP, LW), _f32),
            pltpu.SemaphoreType.DMA,
            pltpu.SemaphoreType.DMA,
        ],
    )
    def k(src_hbm, dst_hbm, s_hbm, d_hbm, c_hbm, ex_hbm, den_hbm,
          sidx, didx, srow, drow, exbuf, cvec, zbuf, den_sh, sem1, sem2):
        cid = lax.axis_index("c")
        sid = lax.axis_index("s")

        @pl.loop(0, ZROWS)
        def _(r):
            zbuf[r, :] = jnp.zeros((LW,), _f32)

        @pl.loop(0, RPT // ZROWS)
        def _(j):
            pltpu.sync_copy(zbuf, den_sh.at[pl.ds(sid * RPT + j * ZROWS, ZROWS)])

        pltpu.sync_copy(c_hbm, cvec)
        plsc.subcore_barrier()
        cv = cvec[...]
        base0 = cid * EPC + sid * EPT

        @pl.loop(0, NCHUNK)
        def _(i):
            base = base0 + i * CH
            pltpu.sync_copy(src_hbm.at[pl.ds(base, CH)], sidx)
            pltpu.sync_copy(dst_hbm.at[pl.ds(base, CH)], didx)
            cp1 = pltpu.async_copy(s_hbm.at[sidx], srow, sem1)
            cp2 = pltpu.async_copy(d_hbm.at[didx], drow, sem2)
            cp1.wait()
            cp2.wait()

            @pl.loop(0, CH)
            def _(r):
                v = srow[r, :] + drow[r, :]
                a = jnp.where(v >= 0.0, v, 0.2 * v)
                exbuf[r, :] = jnp.exp(a - cv)

            pltpu.sync_copy(exbuf, ex_hbm.at[pl.ds(base, CH)])
            pltpu.sync_copy(exbuf, den_sh.at[didx], add=True)

        plsc.subcore_barrier()
        r0 = sid * RPT
        pltpu.sync_copy(den_sh.at[pl.ds(r0, RPT)],
                        den_hbm.at[cid].at[pl.ds(r0, RPT)])

    return k(srcv, dstv, s_tab, d_tab, cvec_hbm)


def _sc_pass2(srcv, dstv, h_tab, ex_hbm_arr, rec_tab, heads8):
    @functools.partial(
        pl.kernel,
        mesh=_sc_mesh(),
        compiler_params=pltpu.CompilerParams(use_tc_tiling_on_sc=False),
        out_type=jax.ShapeDtypeStruct((NC, NP, NH), _f32),
        scratch_types=[
            pltpu.VMEM((CH,), jnp.int32),
            pltpu.VMEM((CH,), jnp.int32),
            pltpu.VMEM((CH, NH), _f32),
            pltpu.VMEM((CH, LW), _f32),
            pltpu.VMEM((CH, LW), _f32),
            pltpu.VMEM((CH, NH), _f32),
            pltpu.VMEM((ZROWS, NH), _f32),
            pltpu.VMEM_SHARED((NP, NH), _f32),
            pltpu.SemaphoreType.DMA,
            pltpu.SemaphoreType.DMA,
        ],
    )
    def k(src_hbm, dst_hbm, h_hbm, ex_hbm, rec_hbm, out_hbm,
          sidx, didx, hrow, exbuf, recrow, msg, zbuf, out_sh, sem1, sem2):
        cid = lax.axis_index("c")
        sid = lax.axis_index("s")

        @pl.loop(0, ZROWS)
        def _(r):
            for j in range(NH // LW):
                zbuf[r, pl.ds(j * LW, LW)] = jnp.zeros((LW,), _f32)

        @pl.loop(0, RPT // ZROWS)
        def _(j):
            pltpu.sync_copy(zbuf, out_sh.at[pl.ds(sid * RPT + j * ZROWS, ZROWS)])

        plsc.subcore_barrier()
        iot = lax.iota(jnp.int32, LW)
        repidx = (iot % 8 if heads8 else iot * 0).reshape(LW, 1)
        dnums = lax.GatherDimensionNumbers(
            offset_dims=(), collapsed_slice_dims=(0,), start_index_map=(0,))
        base0 = cid * EPC + sid * EPT

        @pl.loop(0, NCHUNK)
        def _(i):
            base = base0 + i * CH
            pltpu.sync_copy(src_hbm.at[pl.ds(base, CH)], sidx)
            pltpu.sync_copy(dst_hbm.at[pl.ds(base, CH)], didx)
            cp1 = pltpu.async_copy(h_hbm.at[sidx], hrow, sem1)
            cp2 = pltpu.async_copy(rec_hbm.at[didx], recrow, sem2)
            pltpu.sync_copy(ex_hbm.at[pl.ds(base, CH)], exbuf)
            cp1.wait()
            cp2.wait()

            @pl.loop(0, CH)
            def _(r):
                att = exbuf[r, :] * recrow[r, :]
                rep = lax.gather(att, repidx, dnums, (1,),
                                 mode=lax.GatherScatterMode.PROMISE_IN_BOUNDS)
                for j in range(NH // LW):
                    msg[r, pl.ds(j * LW, LW)] = hrow[r, pl.ds(j * LW, LW)] * rep

            pltpu.sync_copy(msg, out_sh.at[didx], add=True)

        plsc.subcore_barrier()
        r0 = sid * RPT
        pltpu.sync_copy(out_sh.at[pl.ds(r0, RPT)],
                        out_hbm.at[cid].at[pl.ds(r0, RPT)])

    return k(srcv, dstv, h_tab, ex_hbm_arr, rec_tab)


def kernel(x, edge_index, W1, a_src1, a_dst1, b1, W2, a_src2, a_dst2, b2,
           bn_c1_g, bn_c1_b, bn_c2_g, bn_c2_b, Wl1, bl1, bn1_g, bn1_b,
           Wl2, bl2, bn2_g, bn2_b, Wf, bf):
    srcv = edge_index[0]
    dstv = edge_index[1]

    perm = np.array([(j % 8) * 8 + j // 8 for j in range(NH)])
    w1p = W1[:, perm]
    eye8 = jnp.eye(8, dtype=_f32)
    asrc_p = jnp.concatenate(
        [(a_src1.T[:, :, None] * eye8[None]).reshape(NH, 8),
         jnp.zeros((NH, 8), _f32)], axis=1)
    adst_p = jnp.concatenate(
        [(a_dst1.T[:, :, None] * eye8[None]).reshape(NH, 8),
         jnp.zeros((NH, 8), _f32)], axis=1)

    h1p, s1, d1, c1 = _tc_pre1(x, w1p, asrc_p, adst_p)
    ex1, den1p = _sc_pass1(srcv, dstv, s1, d1, c1.reshape(LW))
    rec1 = _tc_rec(den1p)
    out1p = _sc_pass2(srcv, dstv, h1p, ex1, rec1, heads8=True)

    w2p = W2[perm, :]
    as2p = jnp.concatenate([a_src2.T, jnp.zeros((NH, LW - 1), _f32)], axis=1)
    ad2p = jnp.concatenate([a_dst2.T, jnp.zeros((NH, LW - 1), _f32)], axis=1)
    h2, s2, d2, c2 = _tc_mid(
        out1p, b1[perm].reshape(1, NH), bn_c1_g[perm].reshape(1, NH),
        bn_c1_b[perm].reshape(1, NH), w2p, as2p, ad2p)
    ex2, den2p = _sc_pass1(srcv, dstv, s2, d2, c2.reshape(LW))
    rec2 = _tc_rec(den2p)
    out2p = _sc_pass2(srcv, dstv, h2, ex2, rec2, heads8=False)

    y = _tc_head(
        out2p, b2.reshape(1, NH), bn_c2_g.reshape(1, NH),
        bn_c2_b.reshape(1, NH), Wl1, bl1.reshape(1, NH),
        bn1_g.reshape(1, NH), bn1_b.reshape(1, NH), Wl2,
        bl2.reshape(1, NH), bn2_g.reshape(1, NH), bn2_b.reshape(1, NH),
        Wf, bf.reshape(1, 1))
    return y[:N]


# speedup vs baseline: 60.5895x; 1.9206x over previous
import functools

import jax
import jax.numpy as jnp
import numpy as np
from jax import lax
from jax.experimental import pallas as pl
from jax.experimental.pallas import tpu as pltpu
from jax.experimental.pallas import tpu_sc as plsc

N = 10000
NP = 10240
E = 320000
F_IN = 128
NH = 64
LW = 16
NC = 2
NS = 16
EPC = E // NC
EPT = EPC // NS
CH = 125
NCHUNK = EPT // CH
RPT = NP // NS
ZROWS = 128

_f32 = jnp.float32


def _tc_pre1(x, w1p, asrc_p, adst_p):
    blk = 1000

    def body(x_ref, w_ref, as_ref, ad_ref, h_ref, s_ref, d_ref, c_ref, mx_ref):
        i = pl.program_id(0)
        h = jnp.dot(x_ref[...], w_ref[...], preferred_element_type=_f32)
        h_ref[...] = h
        s = jnp.dot(h, as_ref[...], preferred_element_type=_f32)
        d = jnp.dot(h, ad_ref[...], preferred_element_type=_f32)
        s_ref[...] = s
        d_ref[...] = d
        m = jnp.concatenate([jnp.max(s, axis=0, keepdims=True),
                             jnp.max(d, axis=0, keepdims=True)], axis=0)

        @pl.when(i == 0)
        def _():
            mx_ref[...] = m

        @pl.when(i > 0)
        def _():
            mx_ref[...] = jnp.maximum(mx_ref[...], m)

        c_ref[...] = mx_ref[0:1] + mx_ref[1:2]

    return pl.pallas_call(
        body,
        grid=(N // blk,),
        in_specs=[
            pl.BlockSpec((blk, F_IN), lambda i: (i, 0)),
            pl.BlockSpec((F_IN, NH), lambda i: (0, 0)),
            pl.BlockSpec((NH, LW), lambda i: (0, 0)),
            pl.BlockSpec((NH, LW), lambda i: (0, 0)),
        ],
        out_specs=[
            pl.BlockSpec((blk, NH), lambda i: (i, 0)),
            pl.BlockSpec((blk, LW), lambda i: (i, 0)),
            pl.BlockSpec((blk, LW), lambda i: (i, 0)),
            pl.BlockSpec((1, LW), lambda i: (0, 0)),
        ],
        out_shape=[
            jax.ShapeDtypeStruct((N, NH), _f32),
            jax.ShapeDtypeStruct((N, LW), _f32),
            jax.ShapeDtypeStruct((N, LW), _f32),
            jax.ShapeDtypeStruct((1, LW), _f32),
        ],
        scratch_shapes=[pltpu.VMEM((2, LW), _f32)],
    )(x, w1p, asrc_p, adst_p)


def _tc_mid(out1p, den1p, b1p, g1p, be1p, w2p, as2p, ad2p):
    blk = 1024
    ibn = 1.0 / np.sqrt(1.0 + 1e-5)

    def body(o_ref, dp_ref, b_ref, g_ref, be_ref, w_ref, as_ref, ad_ref,
             h_ref, s_ref, d_ref, c_ref, mx_ref):
        i = pl.program_id(0)
        rec = 1.0 / (dp_ref[0] + dp_ref[1] + 1e-16)
        rec_rep = jnp.concatenate([rec[:, 0:8]] * 8, axis=1)
        z = (o_ref[0] + o_ref[1]) * rec_rep + b_ref[...]
        z = jax.nn.relu(z * ibn * g_ref[...] + be_ref[...])
        h = jnp.dot(z, w_ref[...], preferred_element_type=_f32)
        h_ref[...] = h
        s = jnp.dot(h, as_ref[...], preferred_element_type=_f32)
        d = jnp.dot(h, ad_ref[...], preferred_element_type=_f32)
        s_ref[...] = s
        d_ref[...] = d
        m = jnp.concatenate([jnp.max(s, axis=0, keepdims=True),
                             jnp.max(d, axis=0, keepdims=True)], axis=0)

        @pl.when(i == 0)
        def _():
            mx_ref[...] = m

        @pl.when(i > 0)
        def _():
            mx_ref[...] = jnp.maximum(mx_ref[...], m)

        c_ref[...] = mx_ref[0:1] + mx_ref[1:2]

    return pl.pallas_call(
        body,
        grid=(NP // blk,),
        in_specs=[
            pl.BlockSpec((NC, blk, NH), lambda i: (0, i, 0)),
            pl.BlockSpec((NC, blk, LW), lambda i: (0, i, 0)),
            pl.BlockSpec((1, NH), lambda i: (0, 0)),
            pl.BlockSpec((1, NH), lambda i: (0, 0)),
            pl.BlockSpec((1, NH), lambda i: (0, 0)),
            pl.BlockSpec((NH, NH), lambda i: (0, 0)),
            pl.BlockSpec((NH, LW), lambda i: (0, 0)),
            pl.BlockSpec((NH, LW), lambda i: (0, 0)),
        ],
        out_specs=[
            pl.BlockSpec((blk, NH), lambda i: (i, 0)),
            pl.BlockSpec((blk, LW), lambda i: (i, 0)),
            pl.BlockSpec((blk, LW), lambda i: (i, 0)),
            pl.BlockSpec((1, LW), lambda i: (0, 0)),
        ],
        out_shape=[
            jax.ShapeDtypeStruct((NP, NH), _f32),
            jax.ShapeDtypeStruct((NP, LW), _f32),
            jax.ShapeDtypeStruct((NP, LW), _f32),
            jax.ShapeDtypeStruct((1, LW), _f32),
        ],
        scratch_shapes=[pltpu.VMEM((2, LW), _f32)],
    )(out1p, den1p, b1p, g1p, be1p, w2p, as2p, ad2p)


def _tc_head(out2p, den2p, b2, cg, cb, wl1, bl1, g1, be1, wl2, bl2, g2, be2,
             wf, bf):
    blk = 1024
    ibn = 1.0 / np.sqrt(1.0 + 1e-5)

    def body(o_ref, dp_ref, b2_ref, cg_ref, cb_ref, w1_ref, b1_ref, g1_ref,
             be1_ref, w2_ref, b2b_ref, g2_ref, be2_ref, wf_ref, bf_ref, y_ref):
        rec = 1.0 / (dp_ref[0, :, 0:1] + dp_ref[1, :, 0:1] + 1e-16)
        g = (o_ref[0] + o_ref[1]) * rec + b2_ref[...]
        g = g * ibn * cg_ref[...] + cb_ref[...]
        t = jnp.dot(g, w1_ref[...], preferred_element_type=_f32) + b1_ref[...]
        t = jax.nn.relu(t * ibn * g1_ref[...] + be1_ref[...])
        t = jnp.dot(t, w2_ref[...], preferred_element_type=_f32) + b2b_ref[...]
        t = jax.nn.relu(t * ibn * g2_ref[...] + be2_ref[...])
        y = jnp.dot(t, wf_ref[...], preferred_element_type=_f32) + bf_ref[...]
        y_ref[...] = jax.nn.sigmoid(y)

    vec = lambda: pl.BlockSpec((1, NH), lambda i: (0, 0))
    return pl.pallas_call(
        body,
        grid=(NP // blk,),
        in_specs=[
            pl.BlockSpec((NC, blk, NH), lambda i: (0, i, 0)),
            pl.BlockSpec((NC, blk, LW), lambda i: (0, i, 0)),
            vec(), vec(), vec(),
            pl.BlockSpec((NH, NH), lambda i: (0, 0)),
            vec(), vec(), vec(),
            pl.BlockSpec((NH, NH), lambda i: (0, 0)),
            vec(), vec(), vec(),
            pl.BlockSpec((NH, 1), lambda i: (0, 0)),
            pl.BlockSpec((1, 1), lambda i: (0, 0)),
        ],
        out_specs=pl.BlockSpec((blk, 1), lambda i: (i, 0)),
        out_shape=jax.ShapeDtypeStruct((NP, 1), _f32),
    )(out2p, den2p, b2, cg, cb, wl1, bl1, g1, be1, wl2, bl2, g2, be2, wf, bf)


def _sc_layer(src2d, dst2d, s_tab, d_tab, cvec_hbm, h_tab, heads8):
    @functools.partial(
        pl.kernel,
        mesh=plsc.VectorSubcoreMesh(core_axis_name="c", subcore_axis_name="s"),
        compiler_params=pltpu.CompilerParams(use_tc_tiling_on_sc=False),
        out_type=[
            jax.ShapeDtypeStruct((NC, NP, LW), _f32),
            jax.ShapeDtypeStruct((NC, NP, NH), _f32),
        ],
        scratch_types=[
            pltpu.VMEM((NCHUNK, CH), jnp.int32),
            pltpu.VMEM((NCHUNK, CH), jnp.int32),
            pltpu.VMEM((CH, LW), _f32),
            pltpu.VMEM((CH, LW), _f32),
            pltpu.VMEM((CH, LW), _f32),
            pltpu.VMEM((CH, LW), _f32),
            pltpu.VMEM((CH, NH), _f32),
            pltpu.VMEM((CH, NH), _f32),
            pltpu.VMEM((CH, LW), _f32),
            pltpu.VMEM((CH, LW), _f32),
            pltpu.VMEM((CH, NH), _f32),
            pltpu.VMEM((CH, NH), _f32),
            pltpu.VMEM((LW,), _f32),
            pltpu.VMEM((ZROWS, LW), _f32),
            pltpu.VMEM((ZROWS, NH), _f32),
            pltpu.VMEM_SHARED((NP, LW), _f32),
            pltpu.VMEM_SHARED((NP, NH), _f32),
            pltpu.SemaphoreType.DMA,
            pltpu.SemaphoreType.DMA,
            pltpu.SemaphoreType.DMA,
            pltpu.SemaphoreType.DMA,
        ],
    )
    def k(src_hbm, dst_hbm, s_hbm, d_hbm, c_hbm, h_hbm, den_hbm, out_hbm,
          sidx, didx, srow0, srow1, drow0, drow1, hrow0, hrow1,
          exb0, exb1, msg0, msg1, cvec, zb16, zb64, den_sh, out_sh,
          semg0, semg1, semw0, semw1):
        cid = lax.axis_index("c")
        sid = lax.axis_index("s")
        srow = (srow0, srow1)
        drow = (drow0, drow1)
        hrow = (hrow0, hrow1)
        exb = (exb0, exb1)
        msg = (msg0, msg1)
        semg = (semg0, semg1)
        semw = (semw0, semw1)

        @pl.loop(0, ZROWS)
        def _(r):
            zb16[r, :] = jnp.zeros((LW,), _f32)
            for j in range(NH // LW):
                zb64[r, pl.ds(j * LW, LW)] = jnp.zeros((LW,), _f32)

        @pl.loop(0, RPT // ZROWS)
        def _(j):
            pltpu.sync_copy(zb16, den_sh.at[pl.ds(sid * RPT + j * ZROWS, ZROWS)])
            pltpu.sync_copy(zb64, out_sh.at[pl.ds(sid * RPT + j * ZROWS, ZROWS)])

        w0 = (cid * NS + sid) * NCHUNK
        pltpu.sync_copy(src_hbm.at[pl.ds(w0, NCHUNK)], sidx)
        pltpu.sync_copy(dst_hbm.at[pl.ds(w0, NCHUNK)], didx)
        pltpu.sync_copy(c_hbm, cvec)
        plsc.subcore_barrier()
        cv = cvec[...]
        iot = lax.iota(jnp.int32, LW)
        repidx = (iot % 8 if heads8 else iot * 0).reshape(LW, 1)
        dnums = lax.GatherDimensionNumbers(
            offset_dims=(), collapsed_slice_dims=(0,), start_index_map=(0,))

        def start_gathers(i, b):
            pltpu.async_copy(s_hbm.at[sidx.at[i]], srow[b], semg[b])
            pltpu.async_copy(d_hbm.at[didx.at[i]], drow[b], semg[b])
            pltpu.async_copy(h_hbm.at[sidx.at[i]], hrow[b], semg[b])

        def wait_gathers(i, b):
            pltpu.make_async_copy(s_hbm.at[sidx.at[i]], srow[b], semg[b]).wait()
            pltpu.make_async_copy(d_hbm.at[didx.at[i]], drow[b], semg[b]).wait()
            pltpu.make_async_copy(h_hbm.at[sidx.at[i]], hrow[b], semg[b]).wait()

        def start_writes(i, b):
            pltpu.async_copy(exb[b], den_sh.at[didx.at[i]], semw[b], add=True)
            pltpu.async_copy(msg[b], out_sh.at[didx.at[i]], semw[b], add=True)

        def wait_writes(i, b):
            pltpu.make_async_copy(exb[b], den_sh.at[didx.at[i]], semw[b]).wait()
            pltpu.make_async_copy(msg[b], out_sh.at[didx.at[i]], semw[b]).wait()

        start_gathers(0, 0)
        start_gathers(1, 1)

        @pl.loop(0, NCHUNK // 2)
        def _(t):
            for b in range(2):
                i = t * 2 + b
                wait_gathers(i, b)

                @pl.when(i >= 2)
                def _():
                    wait_writes(i - 2, b)

                @pl.loop(0, CH)
                def _(r):
                    v = srow[b][r, :] + drow[b][r, :]
                    a = jnp.where(v >= 0.0, v, 0.2 * v)
                    e = jnp.exp(a - cv)
                    exb[b][r, :] = e
                    rep = lax.gather(e, repidx, dnums, (1,),
                                     mode=lax.GatherScatterMode.PROMISE_IN_BOUNDS)
                    for j in range(NH // LW):
                        msg[b][r, pl.ds(j * LW, LW)] = (
                            hrow[b][r, pl.ds(j * LW, LW)] * rep)

                start_writes(i, b)

                @pl.when(i + 2 < NCHUNK)
                def _():
                    start_gathers(i + 2, b)

        wait_writes(NCHUNK - 2, 0)
        wait_writes(NCHUNK - 1, 1)
        plsc.subcore_barrier()
        r0 = sid * RPT
        pltpu.sync_copy(den_sh.at[pl.ds(r0, RPT)],
                        den_hbm.at[cid].at[pl.ds(r0, RPT)])
        pltpu.sync_copy(out_sh.at[pl.ds(r0, RPT)],
                        out_hbm.at[cid].at[pl.ds(r0, RPT)])

    return k(src2d, dst2d, s_tab, d_tab, cvec_hbm, h_tab)


def kernel(x, edge_index, W1, a_src1, a_dst1, b1, W2, a_src2, a_dst2, b2,
           bn_c1_g, bn_c1_b, bn_c2_g, bn_c2_b, Wl1, bl1, bn1_g, bn1_b,
           Wl2, bl2, bn2_g, bn2_b, Wf, bf):
    src2d = edge_index[0].reshape(E // CH, CH)
    dst2d = edge_index[1].reshape(E // CH, CH)

    perm = np.array([(j % 8) * 8 + j // 8 for j in range(NH)])
    w1p = W1[:, perm]
    eye8 = jnp.eye(8, dtype=_f32)
    asrc_p = jnp.concatenate(
        [(a_src1.T[:, :, None] * eye8[None]).reshape(NH, 8),
         jnp.zeros((NH, 8), _f32)], axis=1)
    adst_p = jnp.concatenate(
        [(a_dst1.T[:, :, None] * eye8[None]).reshape(NH, 8),
         jnp.zeros((NH, 8), _f32)], axis=1)

    h1p, s1, d1, c1 = _tc_pre1(x, w1p, asrc_p, adst_p)
    den1p, out1p = _sc_layer(src2d, dst2d, s1, d1, c1.reshape(LW), h1p,
                             heads8=True)

    w2p = W2[perm, :]
    as2p = jnp.concatenate([a_src2.T, jnp.zeros((NH, LW - 1), _f32)], axis=1)
    ad2p = jnp.concatenate([a_dst2.T, jnp.zeros((NH, LW - 1), _f32)], axis=1)
    h2, s2, d2, c2 = _tc_mid(
        out1p, den1p, b1[perm].reshape(1, NH), bn_c1_g[perm].reshape(1, NH),
        bn_c1_b[perm].reshape(1, NH), w2p, as2p, ad2p)
    den2p, out2p = _sc_layer(src2d, dst2d, s2, d2, c2.reshape(LW), h2,
                             heads8=False)

    y = _tc_head(
        out2p, den2p, b2.reshape(1, NH), bn_c2_g.reshape(1, NH),
        bn_c2_b.reshape(1, NH), Wl1, bl1.reshape(1, NH),
        bn1_g.reshape(1, NH), bn1_b.reshape(1, NH), Wl2,
        bl2.reshape(1, NH), bn2_g.reshape(1, NH), bn2_b.reshape(1, NH),
        Wf, bf.reshape(1, 1))
    return y[:N]
